# baseline (speedup 1.0000x reference)
import functools

import jax
import jax.numpy as jnp
from jax import lax
from jax.experimental import pallas as pl
from jax.experimental.pallas import tpu as pltpu
from jax.experimental.pallas import tpu_sc as plsc

GAMMA = 12.0

NUM_ENTS = 1000000
DIM = 64
BATCH = 4096
NEG = 256

NUM_WORKERS = 32
BPW = BATCH // NUM_WORKERS
RPC = NEG
NCHUNKS = BPW
NGATH = RPC // 128

TCD = 512
NTCB = (NUM_ENTS + TCD - 1) // TCD
NPAIR_ROWS = NTCB * (TCD // 2)


def _tc_relayout_body(src_ref, dst_ref):
  t = jnp.transpose(src_ref[...], (1, 0))
  dst_ref[...] = jnp.concatenate([t[:TCD // 2], t[TCD // 2:]], axis=1)


def _tc_relayout(ent_t):
  return pl.pallas_call(
      _tc_relayout_body,
      grid=(NTCB,),
      in_specs=[pl.BlockSpec((DIM, TCD), lambda j: (0, j))],
      out_specs=pl.BlockSpec((TCD // 2, 2 * DIM), lambda j: (j, 0)),
      out_shape=jax.ShapeDtypeStruct((NPAIR_ROWS, 2 * DIM), jnp.float32),
  )(ent_t)


def _sc_body(x_hbm, tgt_hbm, pos_hbm, neg_hbm, ent_hbm,
             pos_out, neg_out,
             tgt_v, posidx_v, posofs_v, negidx0, negidx1, negofs0, negofs1,
             pred_v, posrow_v, negrow0, negrow1, posout_v, negout0, negout1,
             isem0, isem1, rsem0, rsem1, osem0, osem1):
  nc = 2
  wid = lax.axis_index("s") * nc + lax.axis_index("c")
  base = wid * BPW
  iota = lax.iota(jnp.int32, 16)

  negidx = (negidx0, negidx1)
  negofs = (negofs0, negofs1)
  negrow = (negrow0, negrow1)
  negout = (negout0, negout1)
  isem = (isem0, isem1)
  rsem = (rsem0, rsem1)
  osem = (osem0, osem1)

  def transform(idx_ref, ofs_ref, n):
    def step(k, carry):
      v = idx_ref[pl.ds(k * 16, 16)]
      row = jnp.bitwise_or(
          jnp.bitwise_and(jnp.right_shift(v, 1), -256),
          jnp.bitwise_and(v, 255))
      idx_ref[pl.ds(k * 16, 16)] = row
      ofs_ref[pl.ds(k * 16, 16)] = jnp.bitwise_and(jnp.right_shift(v, 2), 64)
      return carry

    lax.fori_loop(0, n // 16, step, 0)

  def copy_idx(c, buf, sem=None):
    src = neg_hbm.at[base + c]
    dst = negidx[buf]
    if sem is None:
      pltpu.sync_copy(src, dst)
    else:
      pltpu.async_copy(src, dst, sem)

  def wait_idx(c, buf, sem):
    pltpu.make_async_copy(neg_hbm.at[base + c], negidx[buf], sem).wait()

  def out_slice(c):
    return neg_out.at[pl.ds((base + c) * NEG, RPC)]

  def start_rows(buf):
    for j in range(NGATH):
      pltpu.async_copy(
          ent_hbm.at[negidx[buf].at[pl.ds(j * 128, 128)]],
          negrow[buf].at[pl.ds(j * 128, 128), :],
          rsem[buf])

  def wait_rows(buf):
    for j in range(NGATH):
      pltpu.make_async_copy(
          ent_hbm.at[negidx[buf].at[pl.ds(j * 128, 128)]],
          negrow[buf].at[pl.ds(j * 128, 128), :],
          rsem[buf]).wait()

  pltpu.sync_copy(tgt_hbm.at[pl.ds(base, BPW)], tgt_v)
  pltpu.sync_copy(pos_hbm.at[pl.ds(base, BPW)], posidx_v)
  pltpu.sync_copy(x_hbm.at[tgt_v], pred_v)
  transform(posidx_v, posofs_v, BPW)
  pltpu.sync_copy(ent_hbm.at[posidx_v], posrow_v)

  copy_idx(0, 0)
  transform(negidx0, negofs0, RPC)
  start_rows(0)
  copy_idx(1, 1, isem[1])

  def pos_group(g, carry):
    possub = posrow_v.at[pl.ds(g * 16, 16), :]
    predsub = pred_v.at[pl.ds(g * 16, 16), :]
    pofs = posofs_v[pl.ds(g * 16, 16)]

    def d0_body(d0, acc, possub=possub, predsub=predsub, pofs=pofs):
      diag = jnp.bitwise_and(iota + d0, DIM - 1)
      ev = plsc.load_gather(possub, [iota, diag + pofs])
      pv = plsc.load_gather(predsub, [iota, diag])
      return acc + jnp.abs(ev - pv)

    acc = lax.fori_loop(0, DIM, d0_body, jnp.zeros((16,), jnp.float32),
                        unroll=8)
    posout_v[pl.ds(g * 16, 16)] = GAMMA - acc
    return carry

  lax.fori_loop(0, BPW // 16, pos_group, 0)
  pltpu.sync_copy(posout_v, pos_out.at[pl.ds(base, BPW)])

  def compute_chunk(c, buf):
    pred_row = pred_v.at[c]
    ofs = [negofs[buf][pl.ds(g * 16, 16)] for g in range(16)]

    def d0_body(d0, accs, ofs=ofs, buf=buf):
      diag = jnp.bitwise_and(iota + d0, DIM - 1)
      prot = plsc.load_gather(pred_row, [diag])
      new = []
      for g in range(16):
        sub = negrow[buf].at[pl.ds(g * 16, 16), :]
        ev = plsc.load_gather(sub, [iota, diag + ofs[g]])
        new.append(accs[g] + jnp.abs(ev - prot))
      return tuple(new)

    accs = lax.fori_loop(0, DIM, d0_body,
                         (jnp.zeros((16,), jnp.float32),) * 16,
                         unroll=2)
    for g in range(16):
      negout[buf][pl.ds(g * 16, 16)] = GAMMA - accs[g]

  def handle(c, buf):
    wait_rows(buf)

    nbuf = 1 - buf

    @pl.when(c + 1 < NCHUNKS)
    def _():
      wait_idx(c + 1, nbuf, isem[nbuf])
      transform(negidx[nbuf], negofs[nbuf], RPC)
      start_rows(nbuf)

    @pl.when(c + 2 < NCHUNKS)
    def _():
      copy_idx(c + 2, buf, isem[buf])

    @pl.when(c >= 2)
    def _():
      pltpu.make_async_copy(negout[buf], out_slice(c - 2), osem[buf]).wait()

    compute_chunk(c, buf)
    pltpu.async_copy(negout[buf], out_slice(c), osem[buf])

  def pair_body(p, carry):
    handle(2 * p, 0)
    handle(2 * p + 1, 1)
    return carry

  lax.fori_loop(0, NCHUNKS // 2, pair_body, 0)

  pltpu.make_async_copy(negout[0], out_slice(NCHUNKS - 2), osem[0]).wait()
  pltpu.make_async_copy(negout[1], out_slice(NCHUNKS - 1), osem[1]).wait()


@jax.jit
def _full(x, tgt, pos, neg, ent):
  ent_pairs = _tc_relayout(ent.T)
  mesh = plsc.VectorSubcoreMesh(core_axis_name="c", subcore_axis_name="s")
  f = functools.partial(
      pl.kernel,
      mesh=mesh,
      compiler_params=pltpu.CompilerParams(
          needs_layout_passes=False, use_tc_tiling_on_sc=False),
      out_type=(
          jax.ShapeDtypeStruct((BATCH,), jnp.float32),
          jax.ShapeDtypeStruct((BATCH * NEG,), jnp.float32),
      ),
      scratch_types=[
          pltpu.VMEM((BPW,), jnp.int32),
          pltpu.VMEM((BPW,), jnp.int32),
          pltpu.VMEM((BPW,), jnp.int32),
          pltpu.VMEM((RPC,), jnp.int32),
          pltpu.VMEM((RPC,), jnp.int32),
          pltpu.VMEM((RPC,), jnp.int32),
          pltpu.VMEM((RPC,), jnp.int32),
          pltpu.VMEM((BPW, DIM), jnp.float32),
          pltpu.VMEM((BPW, 2 * DIM), jnp.float32),
          pltpu.VMEM((RPC, 2 * DIM), jnp.float32),
          pltpu.VMEM((RPC, 2 * DIM), jnp.float32),
          pltpu.VMEM((BPW,), jnp.float32),
          pltpu.VMEM((RPC,), jnp.float32),
          pltpu.VMEM((RPC,), jnp.float32),
          pltpu.SemaphoreType.DMA,
          pltpu.SemaphoreType.DMA,
          pltpu.SemaphoreType.DMA,
          pltpu.SemaphoreType.DMA,
          pltpu.SemaphoreType.DMA,
          pltpu.SemaphoreType.DMA,
      ],
  )(_sc_body)
  return f(x, tgt, pos, neg, ent_pairs)


def kernel(x, target_node_idxes, positive_samples, negative_samples,
           ent_embedding):
  tgt = target_node_idxes.astype(jnp.int32)
  pos = positive_samples.astype(jnp.int32)
  neg = negative_samples.astype(jnp.int32)
  pos_l, neg_l = _full(x, tgt, pos, neg, ent_embedding)
  return pos_l[:, None], neg_l.reshape(BATCH, NEG)


# speedup vs baseline: 2.2563x; 2.2563x over previous
import functools

import jax
import jax.numpy as jnp
from jax import lax
from jax.experimental import pallas as pl
from jax.experimental.pallas import tpu as pltpu
from jax.experimental.pallas import tpu_sc as plsc

GAMMA = 12.0

NUM_ENTS = 1000000
DIM = 64
BATCH = 4096
NEG = 256

NUM_WORKERS = 32
BPW = BATCH // NUM_WORKERS
CHUNK = 2
NCHUNKS = BPW // CHUNK
RPC = CHUNK * NEG
NGATH = RPC // 128

TCD = 2048
SPAN = TCD // 2
NTCB = (NUM_ENTS + TCD - 1) // TCD
NROWS64 = NTCB * TCD


def _tc_relayout_body(src_ref, dst_ref):
  eye = jnp.eye(DIM, dtype=jnp.float32)
  t = jax.lax.dot_general(src_ref[...], eye, (((0,), (0,)), ((), ())),
                          preferred_element_type=jnp.float32)
  dst_ref[...] = jnp.concatenate([t[:SPAN], t[SPAN:]], axis=1)


def _tc_relayout(ent_t):
  return pl.pallas_call(
      _tc_relayout_body,
      grid=(NTCB,),
      in_specs=[pl.BlockSpec((DIM, TCD), lambda j: (0, j))],
      out_specs=pl.BlockSpec((SPAN, 2 * DIM), lambda j: (j, 0)),
      out_shape=jax.ShapeDtypeStruct((NTCB * SPAN, 2 * DIM), jnp.float32),
  )(ent_t)


def _sc_body(x_hbm, tgt_hbm, pos_hbm, neg_hbm, ent_hbm,
             pos_out, neg_out,
             tgt_v, posidx_v, negidx0, negidx1, pred_v, posrow_v,
             negrow0, negrow1, posout_v, negout0, negout1,
             isem0, isem1, rsem0, rsem1, osem0, osem1):
  nc = 2
  wid = lax.axis_index("s") * nc + lax.axis_index("c")
  base = wid * BPW
  iota = lax.iota(jnp.int32, 16)

  negidx = (negidx0, negidx1)
  negrow = (negrow0, negrow1)
  negout = (negout0, negout1)
  isem = (isem0, isem1)
  rsem = (rsem0, rsem1)
  osem = (osem0, osem1)

  def transform(idx_ref, n):
    def step(k, carry):
      v = idx_ref[pl.ds(k * 16, 16)]
      row = jnp.bitwise_or(
          jnp.bitwise_or(
              jnp.bitwise_and(v, -TCD),
              jnp.left_shift(jnp.bitwise_and(v, SPAN - 1), 1)),
          jnp.bitwise_and(jnp.right_shift(v, 10), 1))
      idx_ref[pl.ds(k * 16, 16)] = row
      return carry

    lax.fori_loop(0, n // 16, step, 0)

  def copy_idx(c, buf, sem=None):
    for k in range(CHUNK):
      src = neg_hbm.at[base + c * CHUNK + k]
      dst = negidx[buf].at[pl.ds(k * NEG, NEG)]
      if sem is None:
        pltpu.sync_copy(src, dst)
      else:
        pltpu.async_copy(src, dst, sem)

  def wait_idx(c, buf, sem):
    for k in range(CHUNK):
      pltpu.make_async_copy(neg_hbm.at[base + c * CHUNK + k],
                            negidx[buf].at[pl.ds(k * NEG, NEG)], sem).wait()

  def out_slice(c):
    return neg_out.at[pl.ds((base + c * CHUNK) * NEG, RPC)]

  def start_rows(buf):
    for j in range(NGATH):
      pltpu.async_copy(
          ent_hbm.at[negidx[buf].at[pl.ds(j * 128, 128)]],
          negrow[buf].at[pl.ds(j * 128, 128), :],
          rsem[buf])

  def wait_rows(buf):
    for j in range(NGATH):
      pltpu.make_async_copy(
          ent_hbm.at[negidx[buf].at[pl.ds(j * 128, 128)]],
          negrow[buf].at[pl.ds(j * 128, 128), :],
          rsem[buf]).wait()

  pltpu.sync_copy(tgt_hbm.at[pl.ds(base, BPW)], tgt_v)
  pltpu.sync_copy(pos_hbm.at[pl.ds(base, BPW)], posidx_v)
  pltpu.sync_copy(x_hbm.at[tgt_v], pred_v)
  transform(posidx_v, BPW)
  pltpu.sync_copy(ent_hbm.at[posidx_v], posrow_v)

  copy_idx(0, 0)
  transform(negidx0, RPC)
  start_rows(0)
  copy_idx(1, 1, isem[1])

  def pos_group(g, carry):
    possub = posrow_v.at[pl.ds(g * 16, 16), :]
    predsub = pred_v.at[pl.ds(g * 16, 16), :]

    def d0_body(d0, acc, possub=possub, predsub=predsub):
      cols = jnp.bitwise_and(iota + d0, DIM - 1)
      ev = plsc.load_gather(possub, [iota, cols])
      pv = plsc.load_gather(predsub, [iota, cols])
      return acc + jnp.abs(ev - pv)

    acc = lax.fori_loop(0, DIM, d0_body, jnp.zeros((16,), jnp.float32),
                        unroll=8)
    posout_v[pl.ds(g * 16, 16)] = GAMMA - acc
    return carry

  lax.fori_loop(0, BPW // 16, pos_group, 0)
  pltpu.sync_copy(posout_v, pos_out.at[pl.ds(base, BPW)])

  def compute_chunk(c, buf):
    for bb in range(CHUNK):
      prow = c * CHUNK + bb
      pred_row = pred_v.at[prow]

      def d0_body(d0, accs, pred_row=pred_row, bb=bb, buf=buf):
        cols = jnp.bitwise_and(iota + d0, DIM - 1)
        prot = plsc.load_gather(pred_row, [cols])
        new = []
        for g in range(16):
          sub = negrow[buf].at[pl.ds(bb * NEG + g * 16, 16), :]
          ev = plsc.load_gather(sub, [iota, cols])
          new.append(accs[g] + jnp.abs(ev - prot))
        return tuple(new)

      accs = lax.fori_loop(0, DIM, d0_body,
                           (jnp.zeros((16,), jnp.float32),) * 16,
                           unroll=4)
      for g in range(16):
        negout[buf][pl.ds(bb * NEG + g * 16, 16)] = GAMMA - accs[g]

  def handle(c, buf):
    wait_rows(buf)

    nbuf = 1 - buf

    @pl.when(c + 1 < NCHUNKS)
    def _():
      wait_idx(c + 1, nbuf, isem[nbuf])
      transform(negidx[nbuf], RPC)
      start_rows(nbuf)

    @pl.when(c + 2 < NCHUNKS)
    def _():
      copy_idx(c + 2, buf, isem[buf])

    @pl.when(c >= 2)
    def _():
      pltpu.make_async_copy(negout[buf], out_slice(c - 2), osem[buf]).wait()

    compute_chunk(c, buf)
    pltpu.async_copy(negout[buf], out_slice(c), osem[buf])

  def pair_body(p, carry):
    handle(2 * p, 0)
    handle(2 * p + 1, 1)
    return carry

  lax.fori_loop(0, NCHUNKS // 2, pair_body, 0)

  pltpu.make_async_copy(negout[0], out_slice(NCHUNKS - 2), osem[0]).wait()
  pltpu.make_async_copy(negout[1], out_slice(NCHUNKS - 1), osem[1]).wait()


@jax.jit
def _full(x, tgt, pos, neg, ent):
  ent_rows = _tc_relayout(ent.T).reshape(NROWS64, DIM)
  mesh = plsc.VectorSubcoreMesh(core_axis_name="c", subcore_axis_name="s")
  f = functools.partial(
      pl.kernel,
      mesh=mesh,
      compiler_params=pltpu.CompilerParams(
          needs_layout_passes=False, use_tc_tiling_on_sc=False),
      out_type=(
          jax.ShapeDtypeStruct((BATCH,), jnp.float32),
          jax.ShapeDtypeStruct((BATCH * NEG,), jnp.float32),
      ),
      scratch_types=[
          pltpu.VMEM((BPW,), jnp.int32),
          pltpu.VMEM((BPW,), jnp.int32),
          pltpu.VMEM((RPC,), jnp.int32),
          pltpu.VMEM((RPC,), jnp.int32),
          pltpu.VMEM((BPW, DIM), jnp.float32),
          pltpu.VMEM((BPW, DIM), jnp.float32),
          pltpu.VMEM((RPC, DIM), jnp.float32),
          pltpu.VMEM((RPC, DIM), jnp.float32),
          pltpu.VMEM((BPW,), jnp.float32),
          pltpu.VMEM((RPC,), jnp.float32),
          pltpu.VMEM((RPC,), jnp.float32),
          pltpu.SemaphoreType.DMA,
          pltpu.SemaphoreType.DMA,
          pltpu.SemaphoreType.DMA,
          pltpu.SemaphoreType.DMA,
          pltpu.SemaphoreType.DMA,
          pltpu.SemaphoreType.DMA,
      ],
  )(_sc_body)
  return f(x, tgt, pos, neg, ent_rows)


def kernel(x, target_node_idxes, positive_samples, negative_samples,
           ent_embedding):
  tgt = target_node_idxes.astype(jnp.int32)
  pos = positive_samples.astype(jnp.int32)
  neg = negative_samples.astype(jnp.int32)
  pos_l, neg_l = _full(x, tgt, pos, neg, ent_embedding)
  return pos_l[:, None], neg_l.reshape(BATCH, NEG)


# speedup vs baseline: 2.2624x; 1.0027x over previous
import functools

import jax
import jax.numpy as jnp
from jax import lax
from jax.experimental import pallas as pl
from jax.experimental.pallas import tpu as pltpu
from jax.experimental.pallas import tpu_sc as plsc

GAMMA = 12.0

NUM_ENTS = 1000000
DIM = 64
BATCH = 4096
NEG = 256

NUM_WORKERS = 32
BPW = BATCH // NUM_WORKERS
CHUNK = 2
NCHUNKS = BPW // CHUNK
RPC = CHUNK * NEG
NGATH = RPC // 128

TCD = 2048
SPAN = TCD // 2
NTCB = (NUM_ENTS + TCD - 1) // TCD
NROWS64 = NTCB * TCD


def _tc_relayout_body(src_ref, dst_ref):
  eye = jnp.eye(DIM, dtype=jnp.float32)
  for h in range(2):
    t = jax.lax.dot_general(
        src_ref[:, pl.ds(h * SPAN, SPAN)], eye, (((0,), (0,)), ((), ())),
        preferred_element_type=jnp.float32)
    dst_ref[:, pl.ds(h * DIM, DIM)] = t


def _tc_relayout(ent_t):
  return pl.pallas_call(
      _tc_relayout_body,
      grid=(NTCB,),
      in_specs=[pl.BlockSpec((DIM, TCD), lambda j: (0, j))],
      out_specs=pl.BlockSpec((SPAN, 2 * DIM), lambda j: (j, 0)),
      out_shape=jax.ShapeDtypeStruct((NTCB * SPAN, 2 * DIM), jnp.float32),
  )(ent_t)


def _sc_body(x_hbm, tgt_hbm, pos_hbm, neg_hbm, ent_hbm,
             pos_out, neg_out,
             tgt_v, posidx_v, negidx0, negidx1, pred_v, posrow_v,
             negrow0, negrow1, posout_v, negout0, negout1,
             isem0, isem1, rsem0, rsem1, osem0, osem1):
  nc = 2
  wid = lax.axis_index("s") * nc + lax.axis_index("c")
  base = wid * BPW
  iota = lax.iota(jnp.int32, 16)

  negidx = (negidx0, negidx1)
  negrow = (negrow0, negrow1)
  negout = (negout0, negout1)
  isem = (isem0, isem1)
  rsem = (rsem0, rsem1)
  osem = (osem0, osem1)

  def transform(idx_ref, n):
    def step(k, carry):
      v = idx_ref[pl.ds(k * 16, 16)]
      row = jnp.bitwise_or(
          jnp.bitwise_or(
              jnp.bitwise_and(v, -TCD),
              jnp.left_shift(jnp.bitwise_and(v, SPAN - 1), 1)),
          jnp.bitwise_and(jnp.right_shift(v, 10), 1))
      idx_ref[pl.ds(k * 16, 16)] = row
      return carry

    lax.fori_loop(0, n // 16, step, 0)

  def copy_idx(c, buf, sem=None):
    for k in range(CHUNK):
      src = neg_hbm.at[base + c * CHUNK + k]
      dst = negidx[buf].at[pl.ds(k * NEG, NEG)]
      if sem is None:
        pltpu.sync_copy(src, dst)
      else:
        pltpu.async_copy(src, dst, sem)

  def wait_idx(c, buf, sem):
    for k in range(CHUNK):
      pltpu.make_async_copy(neg_hbm.at[base + c * CHUNK + k],
                            negidx[buf].at[pl.ds(k * NEG, NEG)], sem).wait()

  def out_slice(c):
    return neg_out.at[pl.ds((base + c * CHUNK) * NEG, RPC)]

  def start_rows(buf):
    for j in range(NGATH):
      pltpu.async_copy(
          ent_hbm.at[negidx[buf].at[pl.ds(j * 128, 128)]],
          negrow[buf].at[pl.ds(j * 128, 128), :],
          rsem[buf])

  def wait_rows(buf):
    for j in range(NGATH):
      pltpu.make_async_copy(
          ent_hbm.at[negidx[buf].at[pl.ds(j * 128, 128)]],
          negrow[buf].at[pl.ds(j * 128, 128), :],
          rsem[buf]).wait()

  pltpu.sync_copy(tgt_hbm.at[pl.ds(base, BPW)], tgt_v)
  pltpu.sync_copy(pos_hbm.at[pl.ds(base, BPW)], posidx_v)
  pltpu.sync_copy(x_hbm.at[tgt_v], pred_v)
  transform(posidx_v, BPW)
  pltpu.sync_copy(ent_hbm.at[posidx_v], posrow_v)

  copy_idx(0, 0)
  transform(negidx0, RPC)
  start_rows(0)
  copy_idx(1, 1, isem[1])

  def pos_group(g, carry):
    possub = posrow_v.at[pl.ds(g * 16, 16), :]
    predsub = pred_v.at[pl.ds(g * 16, 16), :]

    def d0_body(d0, acc, possub=possub, predsub=predsub):
      cols = jnp.bitwise_and(iota + d0, DIM - 1)
      ev = plsc.load_gather(possub, [iota, cols])
      pv = plsc.load_gather(predsub, [iota, cols])
      return acc + jnp.abs(ev - pv)

    acc = lax.fori_loop(0, DIM, d0_body, jnp.zeros((16,), jnp.float32),
                        unroll=8)
    posout_v[pl.ds(g * 16, 16)] = GAMMA - acc
    return carry

  lax.fori_loop(0, BPW // 16, pos_group, 0)
  pltpu.sync_copy(posout_v, pos_out.at[pl.ds(base, BPW)])

  def compute_chunk(c, buf):
    for bb in range(CHUNK):
      prow = c * CHUNK + bb
      pred_row = pred_v.at[prow]

      def d0_body(d0, accs, pred_row=pred_row, bb=bb, buf=buf):
        cols = jnp.bitwise_and(iota + d0, DIM - 1)
        prot = plsc.load_gather(pred_row, [cols])
        new = []
        for g in range(16):
          sub = negrow[buf].at[pl.ds(bb * NEG + g * 16, 16), :]
          ev = plsc.load_gather(sub, [iota, cols])
          new.append(accs[g] + jnp.abs(ev - prot))
        return tuple(new)

      accs = lax.fori_loop(0, DIM, d0_body,
                           (jnp.zeros((16,), jnp.float32),) * 16,
                           unroll=4)
      for g in range(16):
        negout[buf][pl.ds(bb * NEG + g * 16, 16)] = GAMMA - accs[g]

  def handle(c, buf):
    wait_rows(buf)

    nbuf = 1 - buf

    @pl.when(c + 1 < NCHUNKS)
    def _():
      wait_idx(c + 1, nbuf, isem[nbuf])
      transform(negidx[nbuf], RPC)
      start_rows(nbuf)

    @pl.when(c + 2 < NCHUNKS)
    def _():
      copy_idx(c + 2, buf, isem[buf])

    @pl.when(c >= 2)
    def _():
      pltpu.make_async_copy(negout[buf], out_slice(c - 2), osem[buf]).wait()

    compute_chunk(c, buf)
    pltpu.async_copy(negout[buf], out_slice(c), osem[buf])

  def pair_body(p, carry):
    handle(2 * p, 0)
    handle(2 * p + 1, 1)
    return carry

  lax.fori_loop(0, NCHUNKS // 2, pair_body, 0)

  pltpu.make_async_copy(negout[0], out_slice(NCHUNKS - 2), osem[0]).wait()
  pltpu.make_async_copy(negout[1], out_slice(NCHUNKS - 1), osem[1]).wait()


@jax.jit
def _full(x, tgt, pos, neg, ent):
  ent_rows = _tc_relayout(ent.T).reshape(NROWS64, DIM)
  mesh = plsc.VectorSubcoreMesh(core_axis_name="c", subcore_axis_name="s")
  f = functools.partial(
      pl.kernel,
      mesh=mesh,
      compiler_params=pltpu.CompilerParams(
          needs_layout_passes=False, use_tc_tiling_on_sc=False),
      out_type=(
          jax.ShapeDtypeStruct((BATCH,), jnp.float32),
          jax.ShapeDtypeStruct((BATCH * NEG,), jnp.float32),
      ),
      scratch_types=[
          pltpu.VMEM((BPW,), jnp.int32),
          pltpu.VMEM((BPW,), jnp.int32),
          pltpu.VMEM((RPC,), jnp.int32),
          pltpu.VMEM((RPC,), jnp.int32),
          pltpu.VMEM((BPW, DIM), jnp.float32),
          pltpu.VMEM((BPW, DIM), jnp.float32),
          pltpu.VMEM((RPC, DIM), jnp.float32),
          pltpu.VMEM((RPC, DIM), jnp.float32),
          pltpu.VMEM((BPW,), jnp.float32),
          pltpu.VMEM((RPC,), jnp.float32),
          pltpu.VMEM((RPC,), jnp.float32),
          pltpu.SemaphoreType.DMA,
          pltpu.SemaphoreType.DMA,
          pltpu.SemaphoreType.DMA,
          pltpu.SemaphoreType.DMA,
          pltpu.SemaphoreType.DMA,
          pltpu.SemaphoreType.DMA,
      ],
  )(_sc_body)
  return f(x, tgt, pos, neg, ent_rows)


def kernel(x, target_node_idxes, positive_samples, negative_samples,
           ent_embedding):
  tgt = target_node_idxes.astype(jnp.int32)
  pos = positive_samples.astype(jnp.int32)
  neg = negative_samples.astype(jnp.int32)
  pos_l, neg_l = _full(x, tgt, pos, neg, ent_embedding)
  return pos_l[:, None], neg_l.reshape(BATCH, NEG)


# speedup vs baseline: 3.2358x; 1.4302x over previous
import functools

import jax
import jax.numpy as jnp
from jax import lax
from jax.experimental import pallas as pl
from jax.experimental.pallas import tpu as pltpu
from jax.experimental.pallas import tpu_sc as plsc

GAMMA = 12.0

NUM_ENTS = 1000000
DIM = 64
BATCH = 4096
NEG = 256

NUM_WORKERS = 32
BPW = BATCH // NUM_WORKERS
CHUNK = 2
NCHUNKS = BPW // CHUNK
RPC = CHUNK * NEG
NGATH = RPC // 128

TCD = 8192
SPAN = TCD // 2
NTCB = (NUM_ENTS + TCD - 1) // TCD
NROWS64 = NTCB * TCD


def _tc_relayout_body(src_ref, dst_ref):
  eye = jnp.eye(DIM, dtype=jnp.float32)
  for h in range(2):
    t = jax.lax.dot_general(
        src_ref[:, pl.ds(h * SPAN, SPAN)], eye, (((0,), (0,)), ((), ())),
        preferred_element_type=jnp.float32)
    dst_ref[:, pl.ds(h * DIM, DIM)] = t


def _tc_relayout(ent_t):
  return pl.pallas_call(
      _tc_relayout_body,
      grid=(NTCB,),
      in_specs=[pl.BlockSpec((DIM, TCD), lambda j: (0, j))],
      out_specs=pl.BlockSpec((SPAN, 2 * DIM), lambda j: (j, 0)),
      out_shape=jax.ShapeDtypeStruct((NTCB * SPAN, 2 * DIM), jnp.float32),
  )(ent_t)


def _sc_body(x_hbm, tgt_hbm, pos_hbm, neg_hbm, ent_hbm,
             pos_out, neg_out,
             tgt_v, posidx_v, negidx0, negidx1, pred_v, posrow_v,
             negrow0, negrow1, posout_v, negout0, negout1,
             isem0, isem1, rsem0, rsem1, osem0, osem1):
  nc = 2
  wid = lax.axis_index("s") * nc + lax.axis_index("c")
  base = wid * BPW
  iota = lax.iota(jnp.int32, 16)

  negidx = (negidx0, negidx1)
  negrow = (negrow0, negrow1)
  negout = (negout0, negout1)
  isem = (isem0, isem1)
  rsem = (rsem0, rsem1)
  osem = (osem0, osem1)

  def transform(idx_ref, n):
    def step(k, carry):
      v = idx_ref[pl.ds(k * 16, 16)]
      row = jnp.bitwise_or(
          jnp.bitwise_or(
              jnp.bitwise_and(v, -TCD),
              jnp.left_shift(jnp.bitwise_and(v, SPAN - 1), 1)),
          jnp.bitwise_and(jnp.right_shift(v, SPAN.bit_length() - 1), 1))
      idx_ref[pl.ds(k * 16, 16)] = row
      return carry

    lax.fori_loop(0, n // 16, step, 0)

  def copy_idx(c, buf, sem=None):
    for k in range(CHUNK):
      src = neg_hbm.at[base + c * CHUNK + k]
      dst = negidx[buf].at[pl.ds(k * NEG, NEG)]
      if sem is None:
        pltpu.sync_copy(src, dst)
      else:
        pltpu.async_copy(src, dst, sem)

  def wait_idx(c, buf, sem):
    for k in range(CHUNK):
      pltpu.make_async_copy(neg_hbm.at[base + c * CHUNK + k],
                            negidx[buf].at[pl.ds(k * NEG, NEG)], sem).wait()

  def out_slice(c):
    return neg_out.at[pl.ds((base + c * CHUNK) * NEG, RPC)]

  def start_rows(buf):
    for j in range(NGATH):
      pltpu.async_copy(
          ent_hbm.at[negidx[buf].at[pl.ds(j * 128, 128)]],
          negrow[buf].at[pl.ds(j * 128, 128), :],
          rsem[buf])

  def wait_rows(buf):
    for j in range(NGATH):
      pltpu.make_async_copy(
          ent_hbm.at[negidx[buf].at[pl.ds(j * 128, 128)]],
          negrow[buf].at[pl.ds(j * 128, 128), :],
          rsem[buf]).wait()

  pltpu.sync_copy(tgt_hbm.at[pl.ds(base, BPW)], tgt_v)
  pltpu.sync_copy(pos_hbm.at[pl.ds(base, BPW)], posidx_v)
  pltpu.sync_copy(x_hbm.at[tgt_v], pred_v)
  transform(posidx_v, BPW)
  pltpu.sync_copy(ent_hbm.at[posidx_v], posrow_v)

  copy_idx(0, 0)
  transform(negidx0, RPC)
  start_rows(0)
  copy_idx(1, 1, isem[1])

  def pos_group(g, carry):
    possub = posrow_v.at[pl.ds(g * 16, 16), :]
    predsub = pred_v.at[pl.ds(g * 16, 16), :]

    def d0_body(d0, acc, possub=possub, predsub=predsub):
      cols = jnp.bitwise_and(iota + d0, DIM - 1)
      ev = plsc.load_gather(possub, [iota, cols])
      pv = plsc.load_gather(predsub, [iota, cols])
      return acc + jnp.abs(ev - pv)

    acc = lax.fori_loop(0, DIM, d0_body, jnp.zeros((16,), jnp.float32),
                        unroll=8)
    posout_v[pl.ds(g * 16, 16)] = GAMMA - acc
    return carry

  lax.fori_loop(0, BPW // 16, pos_group, 0)
  pltpu.sync_copy(posout_v, pos_out.at[pl.ds(base, BPW)])

  def compute_chunk(c, buf):
    for bb in range(CHUNK):
      prow = c * CHUNK + bb
      pred_row = pred_v.at[prow]

      def d0_body(d0, accs, pred_row=pred_row, bb=bb, buf=buf):
        cols = jnp.bitwise_and(iota + d0, DIM - 1)
        prot = plsc.load_gather(pred_row, [cols])
        new = []
        for g in range(16):
          sub = negrow[buf].at[pl.ds(bb * NEG + g * 16, 16), :]
          ev = plsc.load_gather(sub, [iota, cols])
          new.append(accs[g] + jnp.abs(ev - prot))
        return tuple(new)

      accs = lax.fori_loop(0, DIM, d0_body,
                           (jnp.zeros((16,), jnp.float32),) * 16,
                           unroll=4)
      for g in range(16):
        negout[buf][pl.ds(bb * NEG + g * 16, 16)] = GAMMA - accs[g]

  def handle(c, buf):
    wait_rows(buf)

    nbuf = 1 - buf

    @pl.when(c + 1 < NCHUNKS)
    def _():
      wait_idx(c + 1, nbuf, isem[nbuf])
      transform(negidx[nbuf], RPC)
      start_rows(nbuf)

    @pl.when(c + 2 < NCHUNKS)
    def _():
      copy_idx(c + 2, buf, isem[buf])

    @pl.when(c >= 2)
    def _():
      pltpu.make_async_copy(negout[buf], out_slice(c - 2), osem[buf]).wait()

    compute_chunk(c, buf)
    pltpu.async_copy(negout[buf], out_slice(c), osem[buf])

  def pair_body(p, carry):
    handle(2 * p, 0)
    handle(2 * p + 1, 1)
    return carry

  lax.fori_loop(0, NCHUNKS // 2, pair_body, 0)

  pltpu.make_async_copy(negout[0], out_slice(NCHUNKS - 2), osem[0]).wait()
  pltpu.make_async_copy(negout[1], out_slice(NCHUNKS - 1), osem[1]).wait()


@jax.jit
def _full(x, tgt, pos, neg, ent):
  ent_rows = _tc_relayout(ent.T).reshape(NROWS64, DIM)
  mesh = plsc.VectorSubcoreMesh(core_axis_name="c", subcore_axis_name="s")
  f = functools.partial(
      pl.kernel,
      mesh=mesh,
      compiler_params=pltpu.CompilerParams(
          needs_layout_passes=False, use_tc_tiling_on_sc=False),
      out_type=(
          jax.ShapeDtypeStruct((BATCH,), jnp.float32),
          jax.ShapeDtypeStruct((BATCH * NEG,), jnp.float32),
      ),
      scratch_types=[
          pltpu.VMEM((BPW,), jnp.int32),
          pltpu.VMEM((BPW,), jnp.int32),
          pltpu.VMEM((RPC,), jnp.int32),
          pltpu.VMEM((RPC,), jnp.int32),
          pltpu.VMEM((BPW, DIM), jnp.float32),
          pltpu.VMEM((BPW, DIM), jnp.float32),
          pltpu.VMEM((RPC, DIM), jnp.float32),
          pltpu.VMEM((RPC, DIM), jnp.float32),
          pltpu.VMEM((BPW,), jnp.float32),
          pltpu.VMEM((RPC,), jnp.float32),
          pltpu.VMEM((RPC,), jnp.float32),
          pltpu.SemaphoreType.DMA,
          pltpu.SemaphoreType.DMA,
          pltpu.SemaphoreType.DMA,
          pltpu.SemaphoreType.DMA,
          pltpu.SemaphoreType.DMA,
          pltpu.SemaphoreType.DMA,
      ],
  )(_sc_body)
  return f(x, tgt, pos, neg, ent_rows)


def kernel(x, target_node_idxes, positive_samples, negative_samples,
           ent_embedding):
  tgt = target_node_idxes.astype(jnp.int32)
  pos = positive_samples.astype(jnp.int32)
  neg = negative_samples.astype(jnp.int32)
  pos_l, neg_l = _full(x, tgt, pos, neg, ent_embedding)
  return pos_l[:, None], neg_l.reshape(BATCH, NEG)


# speedup vs baseline: 3.4663x; 1.0712x over previous
import functools

import jax
import jax.numpy as jnp
from jax import lax
from jax.experimental import pallas as pl
from jax.experimental.pallas import tpu as pltpu
from jax.experimental.pallas import tpu_sc as plsc

GAMMA = 12.0

NUM_ENTS = 1000000
DIM = 64
BATCH = 4096
NEG = 256

NUM_WORKERS = 32
BPW = BATCH // NUM_WORKERS
CHUNK = 2
NCHUNKS = BPW // CHUNK
RPC = CHUNK * NEG
NGATH = RPC // 128

TCD = 16384
SPAN = TCD // 2
NTCB = (NUM_ENTS + TCD - 1) // TCD
NROWS64 = NTCB * TCD


def _tc_relayout_body(src_ref, dst_ref):
  eye = jnp.eye(DIM, dtype=jnp.float32)
  for h in range(2):
    t = jax.lax.dot_general(
        src_ref[:, pl.ds(h * SPAN, SPAN)], eye, (((0,), (0,)), ((), ())),
        preferred_element_type=jnp.float32)
    dst_ref[:, pl.ds(h * DIM, DIM)] = t


def _tc_relayout(ent_t):
  return pl.pallas_call(
      _tc_relayout_body,
      grid=(NTCB,),
      in_specs=[pl.BlockSpec((DIM, TCD), lambda j: (0, j))],
      out_specs=pl.BlockSpec((SPAN, 2 * DIM), lambda j: (j, 0)),
      out_shape=jax.ShapeDtypeStruct((NTCB * SPAN, 2 * DIM), jnp.float32),
  )(ent_t)


def _sc_body(x_hbm, tgt_hbm, pos_hbm, neg_hbm, ent_hbm,
             pos_out, neg_out,
             tgt_v, posidx_v, negidx0, negidx1, pred_v, posrow_v,
             negrow0, negrow1, posout_v, negout0, negout1,
             isem0, isem1, rsem0, rsem1, osem0, osem1):
  nc = 2
  wid = lax.axis_index("s") * nc + lax.axis_index("c")
  base = wid * BPW
  iota = lax.iota(jnp.int32, 16)

  negidx = (negidx0, negidx1)
  negrow = (negrow0, negrow1)
  negout = (negout0, negout1)
  isem = (isem0, isem1)
  rsem = (rsem0, rsem1)
  osem = (osem0, osem1)

  def transform(idx_ref, n):
    def step(k, carry):
      v = idx_ref[pl.ds(k * 16, 16)]
      row = jnp.bitwise_or(
          jnp.bitwise_or(
              jnp.bitwise_and(v, -TCD),
              jnp.left_shift(jnp.bitwise_and(v, SPAN - 1), 1)),
          jnp.bitwise_and(jnp.right_shift(v, SPAN.bit_length() - 1), 1))
      idx_ref[pl.ds(k * 16, 16)] = row
      return carry

    lax.fori_loop(0, n // 16, step, 0)

  def copy_idx(c, buf, sem=None):
    for k in range(CHUNK):
      src = neg_hbm.at[base + c * CHUNK + k]
      dst = negidx[buf].at[pl.ds(k * NEG, NEG)]
      if sem is None:
        pltpu.sync_copy(src, dst)
      else:
        pltpu.async_copy(src, dst, sem)

  def wait_idx(c, buf, sem):
    for k in range(CHUNK):
      pltpu.make_async_copy(neg_hbm.at[base + c * CHUNK + k],
                            negidx[buf].at[pl.ds(k * NEG, NEG)], sem).wait()

  def out_slice(c):
    return neg_out.at[pl.ds((base + c * CHUNK) * NEG, RPC)]

  def start_rows(buf):
    for j in range(NGATH):
      pltpu.async_copy(
          ent_hbm.at[negidx[buf].at[pl.ds(j * 128, 128)]],
          negrow[buf].at[pl.ds(j * 128, 128), :],
          rsem[buf])

  def wait_rows(buf):
    for j in range(NGATH):
      pltpu.make_async_copy(
          ent_hbm.at[negidx[buf].at[pl.ds(j * 128, 128)]],
          negrow[buf].at[pl.ds(j * 128, 128), :],
          rsem[buf]).wait()

  pltpu.sync_copy(tgt_hbm.at[pl.ds(base, BPW)], tgt_v)
  pltpu.sync_copy(pos_hbm.at[pl.ds(base, BPW)], posidx_v)
  pltpu.sync_copy(x_hbm.at[tgt_v], pred_v)
  transform(posidx_v, BPW)
  pltpu.sync_copy(ent_hbm.at[posidx_v], posrow_v)

  copy_idx(0, 0)
  transform(negidx0, RPC)
  start_rows(0)
  copy_idx(1, 1, isem[1])

  def pos_group(g, carry):
    possub = posrow_v.at[pl.ds(g * 16, 16), :]
    predsub = pred_v.at[pl.ds(g * 16, 16), :]

    def d0_body(d0, acc, possub=possub, predsub=predsub):
      cols = jnp.bitwise_and(iota + d0, DIM - 1)
      ev = plsc.load_gather(possub, [iota, cols])
      pv = plsc.load_gather(predsub, [iota, cols])
      return acc + jnp.abs(ev - pv)

    acc = lax.fori_loop(0, DIM, d0_body, jnp.zeros((16,), jnp.float32),
                        unroll=8)
    posout_v[pl.ds(g * 16, 16)] = GAMMA - acc
    return carry

  lax.fori_loop(0, BPW // 16, pos_group, 0)
  pltpu.sync_copy(posout_v, pos_out.at[pl.ds(base, BPW)])

  def compute_chunk(c, buf):
    for bb in range(CHUNK):
      prow = c * CHUNK + bb
      pred_row = pred_v.at[prow]

      def d0_body(d0, accs, pred_row=pred_row, bb=bb, buf=buf):
        cols = jnp.bitwise_and(iota + d0, DIM - 1)
        prot = plsc.load_gather(pred_row, [cols])
        new = []
        for g in range(16):
          sub = negrow[buf].at[pl.ds(bb * NEG + g * 16, 16), :]
          ev = plsc.load_gather(sub, [iota, cols])
          new.append(accs[g] + jnp.abs(ev - prot))
        return tuple(new)

      accs = lax.fori_loop(0, DIM, d0_body,
                           (jnp.zeros((16,), jnp.float32),) * 16,
                           unroll=4)
      for g in range(16):
        negout[buf][pl.ds(bb * NEG + g * 16, 16)] = GAMMA - accs[g]

  def handle(c, buf):
    wait_rows(buf)

    nbuf = 1 - buf

    @pl.when(c + 1 < NCHUNKS)
    def _():
      wait_idx(c + 1, nbuf, isem[nbuf])
      transform(negidx[nbuf], RPC)
      start_rows(nbuf)

    @pl.when(c + 2 < NCHUNKS)
    def _():
      copy_idx(c + 2, buf, isem[buf])

    @pl.when(c >= 2)
    def _():
      pltpu.make_async_copy(negout[buf], out_slice(c - 2), osem[buf]).wait()

    compute_chunk(c, buf)
    pltpu.async_copy(negout[buf], out_slice(c), osem[buf])

  def pair_body(p, carry):
    handle(2 * p, 0)
    handle(2 * p + 1, 1)
    return carry

  lax.fori_loop(0, NCHUNKS // 2, pair_body, 0)

  pltpu.make_async_copy(negout[0], out_slice(NCHUNKS - 2), osem[0]).wait()
  pltpu.make_async_copy(negout[1], out_slice(NCHUNKS - 1), osem[1]).wait()


@jax.jit
def _full(x, tgt, pos, neg, ent):
  ent_rows = _tc_relayout(ent.T).reshape(NROWS64, DIM)
  mesh = plsc.VectorSubcoreMesh(core_axis_name="c", subcore_axis_name="s")
  f = functools.partial(
      pl.kernel,
      mesh=mesh,
      compiler_params=pltpu.CompilerParams(
          needs_layout_passes=False, use_tc_tiling_on_sc=False),
      out_type=(
          jax.ShapeDtypeStruct((BATCH,), jnp.float32),
          jax.ShapeDtypeStruct((BATCH * NEG,), jnp.float32),
      ),
      scratch_types=[
          pltpu.VMEM((BPW,), jnp.int32),
          pltpu.VMEM((BPW,), jnp.int32),
          pltpu.VMEM((RPC,), jnp.int32),
          pltpu.VMEM((RPC,), jnp.int32),
          pltpu.VMEM((BPW, DIM), jnp.float32),
          pltpu.VMEM((BPW, DIM), jnp.float32),
          pltpu.VMEM((RPC, DIM), jnp.float32),
          pltpu.VMEM((RPC, DIM), jnp.float32),
          pltpu.VMEM((BPW,), jnp.float32),
          pltpu.VMEM((RPC,), jnp.float32),
          pltpu.VMEM((RPC,), jnp.float32),
          pltpu.SemaphoreType.DMA,
          pltpu.SemaphoreType.DMA,
          pltpu.SemaphoreType.DMA,
          pltpu.SemaphoreType.DMA,
          pltpu.SemaphoreType.DMA,
          pltpu.SemaphoreType.DMA,
      ],
  )(_sc_body)
  return f(x, tgt, pos, neg, ent_rows)


def kernel(x, target_node_idxes, positive_samples, negative_samples,
           ent_embedding):
  tgt = target_node_idxes.astype(jnp.int32)
  pos = positive_samples.astype(jnp.int32)
  neg = negative_samples.astype(jnp.int32)
  pos_l, neg_l = _full(x, tgt, pos, neg, ent_embedding)
  return pos_l[:, None], neg_l.reshape(BATCH, NEG)


# speedup vs baseline: 3.5904x; 1.0358x over previous
import functools

import jax
import jax.numpy as jnp
from jax import lax
from jax.experimental import pallas as pl
from jax.experimental.pallas import tpu as pltpu
from jax.experimental.pallas import tpu_sc as plsc

GAMMA = 12.0

NUM_ENTS = 1000000
DIM = 64
BATCH = 4096
NEG = 256

NUM_WORKERS = 32
BPW = BATCH // NUM_WORKERS
CHUNK = 2
NCHUNKS = BPW // CHUNK
RPC = CHUNK * NEG
NGATH = RPC // 128

TCD = 32768
SPAN = TCD // 2
NTCB = (NUM_ENTS + TCD - 1) // TCD
NROWS64 = NTCB * TCD


def _tc_relayout_body(src_ref, dst_ref):
  eye = jnp.eye(DIM, dtype=jnp.float32)
  for h in range(2):
    t = jax.lax.dot_general(
        src_ref[:, pl.ds(h * SPAN, SPAN)], eye, (((0,), (0,)), ((), ())),
        preferred_element_type=jnp.float32)
    dst_ref[:, pl.ds(h * DIM, DIM)] = t


def _tc_relayout(ent_t):
  return pl.pallas_call(
      _tc_relayout_body,
      grid=(NTCB,),
      in_specs=[pl.BlockSpec((DIM, TCD), lambda j: (0, j))],
      out_specs=pl.BlockSpec((SPAN, 2 * DIM), lambda j: (j, 0)),
      out_shape=jax.ShapeDtypeStruct((NTCB * SPAN, 2 * DIM), jnp.float32),
  )(ent_t)


def _sc_body(x_hbm, tgt_hbm, pos_hbm, neg_hbm, ent_hbm,
             pos_out, neg_out,
             tgt_v, posidx_v, negidx0, negidx1, pred_v, posrow_v,
             negrow0, negrow1, posout_v, negout0, negout1,
             isem0, isem1, rsem0, rsem1, osem0, osem1):
  nc = 2
  wid = lax.axis_index("s") * nc + lax.axis_index("c")
  base = wid * BPW
  iota = lax.iota(jnp.int32, 16)

  negidx = (negidx0, negidx1)
  negrow = (negrow0, negrow1)
  negout = (negout0, negout1)
  isem = (isem0, isem1)
  rsem = (rsem0, rsem1)
  osem = (osem0, osem1)

  def transform(idx_ref, n):
    def step(k, carry):
      v = idx_ref[pl.ds(k * 16, 16)]
      row = jnp.bitwise_or(
          jnp.bitwise_or(
              jnp.bitwise_and(v, -TCD),
              jnp.left_shift(jnp.bitwise_and(v, SPAN - 1), 1)),
          jnp.bitwise_and(jnp.right_shift(v, SPAN.bit_length() - 1), 1))
      idx_ref[pl.ds(k * 16, 16)] = row
      return carry

    lax.fori_loop(0, n // 16, step, 0)

  def copy_idx(c, buf, sem=None):
    for k in range(CHUNK):
      src = neg_hbm.at[base + c * CHUNK + k]
      dst = negidx[buf].at[pl.ds(k * NEG, NEG)]
      if sem is None:
        pltpu.sync_copy(src, dst)
      else:
        pltpu.async_copy(src, dst, sem)

  def wait_idx(c, buf, sem):
    for k in range(CHUNK):
      pltpu.make_async_copy(neg_hbm.at[base + c * CHUNK + k],
                            negidx[buf].at[pl.ds(k * NEG, NEG)], sem).wait()

  def out_slice(c):
    return neg_out.at[pl.ds((base + c * CHUNK) * NEG, RPC)]

  def start_rows(buf):
    for j in range(NGATH):
      pltpu.async_copy(
          ent_hbm.at[negidx[buf].at[pl.ds(j * 128, 128)]],
          negrow[buf].at[pl.ds(j * 128, 128), :],
          rsem[buf])

  def wait_rows(buf):
    for j in range(NGATH):
      pltpu.make_async_copy(
          ent_hbm.at[negidx[buf].at[pl.ds(j * 128, 128)]],
          negrow[buf].at[pl.ds(j * 128, 128), :],
          rsem[buf]).wait()

  pltpu.sync_copy(tgt_hbm.at[pl.ds(base, BPW)], tgt_v)
  pltpu.sync_copy(pos_hbm.at[pl.ds(base, BPW)], posidx_v)
  pltpu.sync_copy(x_hbm.at[tgt_v], pred_v)
  transform(posidx_v, BPW)
  pltpu.sync_copy(ent_hbm.at[posidx_v], posrow_v)

  copy_idx(0, 0)
  transform(negidx0, RPC)
  start_rows(0)
  copy_idx(1, 1, isem[1])

  def pos_group(g, carry):
    possub = posrow_v.at[pl.ds(g * 16, 16), :]
    predsub = pred_v.at[pl.ds(g * 16, 16), :]

    def d0_body(d0, acc, possub=possub, predsub=predsub):
      cols = jnp.bitwise_and(iota + d0, DIM - 1)
      ev = plsc.load_gather(possub, [iota, cols])
      pv = plsc.load_gather(predsub, [iota, cols])
      return acc + jnp.abs(ev - pv)

    acc = lax.fori_loop(0, DIM, d0_body, jnp.zeros((16,), jnp.float32),
                        unroll=8)
    posout_v[pl.ds(g * 16, 16)] = GAMMA - acc
    return carry

  lax.fori_loop(0, BPW // 16, pos_group, 0)
  pltpu.sync_copy(posout_v, pos_out.at[pl.ds(base, BPW)])

  def compute_chunk(c, buf):
    for bb in range(CHUNK):
      prow = c * CHUNK + bb
      pred_row = pred_v.at[prow]

      def d0_body(d0, accs, pred_row=pred_row, bb=bb, buf=buf):
        cols = jnp.bitwise_and(iota + d0, DIM - 1)
        prot = plsc.load_gather(pred_row, [cols])
        new = []
        for g in range(16):
          sub = negrow[buf].at[pl.ds(bb * NEG + g * 16, 16), :]
          ev = plsc.load_gather(sub, [iota, cols])
          new.append(accs[g] + jnp.abs(ev - prot))
        return tuple(new)

      accs = lax.fori_loop(0, DIM, d0_body,
                           (jnp.zeros((16,), jnp.float32),) * 16,
                           unroll=4)
      for g in range(16):
        negout[buf][pl.ds(bb * NEG + g * 16, 16)] = GAMMA - accs[g]

  def handle(c, buf):
    wait_rows(buf)

    nbuf = 1 - buf

    @pl.when(c + 1 < NCHUNKS)
    def _():
      wait_idx(c + 1, nbuf, isem[nbuf])
      transform(negidx[nbuf], RPC)
      start_rows(nbuf)

    @pl.when(c + 2 < NCHUNKS)
    def _():
      copy_idx(c + 2, buf, isem[buf])

    @pl.when(c >= 2)
    def _():
      pltpu.make_async_copy(negout[buf], out_slice(c - 2), osem[buf]).wait()

    compute_chunk(c, buf)
    pltpu.async_copy(negout[buf], out_slice(c), osem[buf])

  def pair_body(p, carry):
    handle(2 * p, 0)
    handle(2 * p + 1, 1)
    return carry

  lax.fori_loop(0, NCHUNKS // 2, pair_body, 0)

  pltpu.make_async_copy(negout[0], out_slice(NCHUNKS - 2), osem[0]).wait()
  pltpu.make_async_copy(negout[1], out_slice(NCHUNKS - 1), osem[1]).wait()


@jax.jit
def _full(x, tgt, pos, neg, ent):
  ent_rows = _tc_relayout(ent.T).reshape(NROWS64, DIM)
  mesh = plsc.VectorSubcoreMesh(core_axis_name="c", subcore_axis_name="s")
  f = functools.partial(
      pl.kernel,
      mesh=mesh,
      compiler_params=pltpu.CompilerParams(
          needs_layout_passes=False, use_tc_tiling_on_sc=False),
      out_type=(
          jax.ShapeDtypeStruct((BATCH,), jnp.float32),
          jax.ShapeDtypeStruct((BATCH * NEG,), jnp.float32),
      ),
      scratch_types=[
          pltpu.VMEM((BPW,), jnp.int32),
          pltpu.VMEM((BPW,), jnp.int32),
          pltpu.VMEM((RPC,), jnp.int32),
          pltpu.VMEM((RPC,), jnp.int32),
          pltpu.VMEM((BPW, DIM), jnp.float32),
          pltpu.VMEM((BPW, DIM), jnp.float32),
          pltpu.VMEM((RPC, DIM), jnp.float32),
          pltpu.VMEM((RPC, DIM), jnp.float32),
          pltpu.VMEM((BPW,), jnp.float32),
          pltpu.VMEM((RPC,), jnp.float32),
          pltpu.VMEM((RPC,), jnp.float32),
          pltpu.SemaphoreType.DMA,
          pltpu.SemaphoreType.DMA,
          pltpu.SemaphoreType.DMA,
          pltpu.SemaphoreType.DMA,
          pltpu.SemaphoreType.DMA,
          pltpu.SemaphoreType.DMA,
      ],
  )(_sc_body)
  return f(x, tgt, pos, neg, ent_rows)


def kernel(x, target_node_idxes, positive_samples, negative_samples,
           ent_embedding):
  tgt = target_node_idxes.astype(jnp.int32)
  pos = positive_samples.astype(jnp.int32)
  neg = negative_samples.astype(jnp.int32)
  pos_l, neg_l = _full(x, tgt, pos, neg, ent_embedding)
  return pos_l[:, None], neg_l.reshape(BATCH, NEG)


# speedup vs baseline: 3.9305x; 1.0947x over previous
import functools

import jax
import jax.numpy as jnp
from jax import lax
from jax.experimental import pallas as pl
from jax.experimental.pallas import tpu as pltpu
from jax.experimental.pallas import tpu_sc as plsc

GAMMA = 12.0

NUM_ENTS = 1000000
DIM = 64
BATCH = 4096
NEG = 256

NUM_WORKERS = 32
BPW = BATCH // NUM_WORKERS
CHUNK = 2
NCHUNKS = BPW // CHUNK
RPC = CHUNK * NEG
NGATH = RPC // 128

TCD = 32768
SPAN = TCD // 2
NTCB = (NUM_ENTS + TCD - 1) // TCD
NROWS64 = NTCB * TCD


def _tc_relayout_body(src_ref, dst_ref):
  eye = jnp.eye(DIM, dtype=jnp.float32)
  for h in range(2):
    t = jax.lax.dot_general(
        src_ref[:, pl.ds(h * SPAN, SPAN)], eye, (((0,), (0,)), ((), ())),
        preferred_element_type=jnp.float32)
    dst_ref[:, pl.ds(h * DIM, DIM)] = t


def _tc_relayout(ent_t):
  return pl.pallas_call(
      _tc_relayout_body,
      grid=(NTCB,),
      in_specs=[pl.BlockSpec((DIM, TCD), lambda j: (0, j))],
      out_specs=pl.BlockSpec((SPAN, 2 * DIM), lambda j: (j, 0)),
      out_shape=jax.ShapeDtypeStruct((NTCB * SPAN, 2 * DIM), jnp.float32),
  )(ent_t)


def _sc_body(x_hbm, tgt_hbm, pos_hbm, neg_hbm, ent_hbm,
             pos_out, neg_out,
             tgt_v, posidx_v, negidx0, negidx1, pred_v, posrow_v,
             negrow0, negrow1, posout_v, negout0, negout1,
             isem0, isem1, rsem0, rsem1, osem0, osem1):
  nc = 2
  wid = lax.axis_index("s") * nc + lax.axis_index("c")
  base = wid * BPW
  iota = lax.iota(jnp.int32, 16)

  negidx = (negidx0, negidx1)
  negrow = (negrow0, negrow1)
  negout = (negout0, negout1)
  isem = (isem0, isem1)
  rsem = (rsem0, rsem1)
  osem = (osem0, osem1)

  def transform(idx_ref, n):
    def step(k, carry):
      v = idx_ref[pl.ds(k * 16, 16)]
      row = jnp.bitwise_or(
          jnp.bitwise_or(
              jnp.bitwise_and(v, -TCD),
              jnp.left_shift(jnp.bitwise_and(v, SPAN - 1), 1)),
          jnp.bitwise_and(jnp.right_shift(v, SPAN.bit_length() - 1), 1))
      idx_ref[pl.ds(k * 16, 16)] = row
      return carry

    lax.fori_loop(0, n // 16, step, 0)

  def copy_idx(c, buf, sem=None):
    for k in range(CHUNK):
      src = neg_hbm.at[base + c * CHUNK + k]
      dst = negidx[buf].at[pl.ds(k * NEG, NEG)]
      if sem is None:
        pltpu.sync_copy(src, dst)
      else:
        pltpu.async_copy(src, dst, sem)

  def wait_idx(c, buf, sem):
    for k in range(CHUNK):
      pltpu.make_async_copy(neg_hbm.at[base + c * CHUNK + k],
                            negidx[buf].at[pl.ds(k * NEG, NEG)], sem).wait()

  def out_slice(c):
    return neg_out.at[pl.ds((base + c * CHUNK) * NEG, RPC)]

  def start_rows(buf):
    for j in range(NGATH):
      pltpu.async_copy(
          ent_hbm.at[negidx[buf].at[pl.ds(j * 128, 128)]],
          negrow[buf].at[pl.ds(j * 128, 128), :],
          rsem[buf])

  def wait_rows(buf):
    for j in range(NGATH):
      pltpu.make_async_copy(
          ent_hbm.at[negidx[buf].at[pl.ds(j * 128, 128)]],
          negrow[buf].at[pl.ds(j * 128, 128), :],
          rsem[buf]).wait()

  pltpu.sync_copy(tgt_hbm.at[pl.ds(base, BPW)], tgt_v)
  pltpu.sync_copy(pos_hbm.at[pl.ds(base, BPW)], posidx_v)
  pltpu.sync_copy(x_hbm.at[tgt_v], pred_v)
  transform(posidx_v, BPW)
  pltpu.sync_copy(ent_hbm.at[posidx_v], posrow_v)

  copy_idx(0, 0)
  transform(negidx0, RPC)
  start_rows(0)
  copy_idx(1, 1, isem[1])

  def pos_group(g, carry):
    possub = posrow_v.at[pl.ds(g * 16, 16), :]
    predsub = pred_v.at[pl.ds(g * 16, 16), :]

    def d0_body(d0, acc, possub=possub, predsub=predsub):
      cols = jnp.bitwise_and(iota + d0, DIM - 1)
      ev = plsc.load_gather(possub, [iota, cols])
      pv = plsc.load_gather(predsub, [iota, cols])
      return acc + jnp.abs(ev - pv)

    acc = lax.fori_loop(0, DIM, d0_body, jnp.zeros((16,), jnp.float32),
                        unroll=8)
    posout_v[pl.ds(g * 16, 16)] = GAMMA - acc
    return carry

  lax.fori_loop(0, BPW // 16, pos_group, 0)
  pltpu.sync_copy(posout_v, pos_out.at[pl.ds(base, BPW)])

  def compute_chunk(c, buf):
    for bb in range(CHUNK):
      prow = c * CHUNK + bb
      pred_row = pred_v.at[prow]

      def d0_body(d0, accs, pred_row=pred_row, bb=bb, buf=buf):
        cols = jnp.bitwise_and(iota + d0, DIM - 1)
        prot = plsc.load_gather(pred_row, [cols])
        new = []
        for g in range(16):
          sub = negrow[buf].at[pl.ds(bb * NEG + g * 16, 16), :]
          ev = plsc.load_gather(sub, [iota, cols])
          new.append(accs[g] + jnp.abs(ev - prot))
        return tuple(new)

      accs = lax.fori_loop(0, DIM, d0_body,
                           (jnp.zeros((16,), jnp.float32),) * 16,
                           unroll=8)
      for g in range(16):
        negout[buf][pl.ds(bb * NEG + g * 16, 16)] = GAMMA - accs[g]

  def handle(c, buf):
    wait_rows(buf)

    nbuf = 1 - buf

    @pl.when(c + 1 < NCHUNKS)
    def _():
      wait_idx(c + 1, nbuf, isem[nbuf])
      transform(negidx[nbuf], RPC)
      start_rows(nbuf)

    @pl.when(c + 2 < NCHUNKS)
    def _():
      copy_idx(c + 2, buf, isem[buf])

    @pl.when(c >= 2)
    def _():
      pltpu.make_async_copy(negout[buf], out_slice(c - 2), osem[buf]).wait()

    compute_chunk(c, buf)
    pltpu.async_copy(negout[buf], out_slice(c), osem[buf])

  def pair_body(p, carry):
    handle(2 * p, 0)
    handle(2 * p + 1, 1)
    return carry

  lax.fori_loop(0, NCHUNKS // 2, pair_body, 0)

  pltpu.make_async_copy(negout[0], out_slice(NCHUNKS - 2), osem[0]).wait()
  pltpu.make_async_copy(negout[1], out_slice(NCHUNKS - 1), osem[1]).wait()


@jax.jit
def _full(x, tgt, pos, neg, ent):
  ent_rows = _tc_relayout(ent.T).reshape(NROWS64, DIM)
  mesh = plsc.VectorSubcoreMesh(core_axis_name="c", subcore_axis_name="s")
  f = functools.partial(
      pl.kernel,
      mesh=mesh,
      compiler_params=pltpu.CompilerParams(
          needs_layout_passes=False, use_tc_tiling_on_sc=False),
      out_type=(
          jax.ShapeDtypeStruct((BATCH,), jnp.float32),
          jax.ShapeDtypeStruct((BATCH * NEG,), jnp.float32),
      ),
      scratch_types=[
          pltpu.VMEM((BPW,), jnp.int32),
          pltpu.VMEM((BPW,), jnp.int32),
          pltpu.VMEM((RPC,), jnp.int32),
          pltpu.VMEM((RPC,), jnp.int32),
          pltpu.VMEM((BPW, DIM), jnp.float32),
          pltpu.VMEM((BPW, DIM), jnp.float32),
          pltpu.VMEM((RPC, DIM), jnp.float32),
          pltpu.VMEM((RPC, DIM), jnp.float32),
          pltpu.VMEM((BPW,), jnp.float32),
          pltpu.VMEM((RPC,), jnp.float32),
          pltpu.VMEM((RPC,), jnp.float32),
          pltpu.SemaphoreType.DMA,
          pltpu.SemaphoreType.DMA,
          pltpu.SemaphoreType.DMA,
          pltpu.SemaphoreType.DMA,
          pltpu.SemaphoreType.DMA,
          pltpu.SemaphoreType.DMA,
      ],
  )(_sc_body)
  return f(x, tgt, pos, neg, ent_rows)


def kernel(x, target_node_idxes, positive_samples, negative_samples,
           ent_embedding):
  tgt = target_node_idxes.astype(jnp.int32)
  pos = positive_samples.astype(jnp.int32)
  neg = negative_samples.astype(jnp.int32)
  pos_l, neg_l = _full(x, tgt, pos, neg, ent_embedding)
  return pos_l[:, None], neg_l.reshape(BATCH, NEG)


# speedup vs baseline: 3.9314x; 1.0002x over previous
import functools

import jax
import jax.numpy as jnp
from jax import lax
from jax.experimental import pallas as pl
from jax.experimental.pallas import tpu as pltpu
from jax.experimental.pallas import tpu_sc as plsc

GAMMA = 12.0

NUM_ENTS = 1000000
DIM = 64
BATCH = 4096
NEG = 256

NUM_WORKERS = 32
BPW = BATCH // NUM_WORKERS
CHUNK = 2
NCHUNKS = BPW // CHUNK
RPC = CHUNK * NEG
NGATH = RPC // 128

TCD = 32768
SPAN = TCD // 2
NTCB = (NUM_ENTS + TCD - 1) // TCD
NROWS64 = NTCB * TCD


def _tc_relayout_body(src_ref, dst_ref):
  eye = jnp.eye(DIM, dtype=jnp.float32)
  for h in range(2):
    t = jax.lax.dot_general(
        src_ref[:, pl.ds(h * SPAN, SPAN)], eye, (((0,), (0,)), ((), ())),
        preferred_element_type=jnp.float32)
    dst_ref[:, pl.ds(h * DIM, DIM)] = t


def _tc_relayout(ent_t):
  return pl.pallas_call(
      _tc_relayout_body,
      grid=(NTCB,),
      in_specs=[pl.BlockSpec((DIM, TCD), lambda j: (0, j))],
      out_specs=pl.BlockSpec((SPAN, 2 * DIM), lambda j: (j, 0)),
      out_shape=jax.ShapeDtypeStruct((NTCB * SPAN, 2 * DIM), jnp.float32),
  )(ent_t)


def _sc_body(x_hbm, tgt_hbm, pos_hbm, neg_hbm, ent_hbm,
             pos_out, neg_out,
             tgt_v, posidx_v, negidx0, negidx1, pred_v, posrow_v,
             negrow0, negrow1, posout_v, negout0, negout1,
             isem0, isem1, rsem0, rsem1, osem0, osem1):
  nc = 2
  wid = lax.axis_index("s") * nc + lax.axis_index("c")
  base = wid * BPW
  iota = lax.iota(jnp.int32, 16)

  negidx = (negidx0, negidx1)
  negrow = (negrow0, negrow1)
  negout = (negout0, negout1)
  isem = (isem0, isem1)
  rsem = (rsem0, rsem1)
  osem = (osem0, osem1)

  def transform(idx_ref, n):
    def step(k, carry):
      v = idx_ref[pl.ds(k * 16, 16)]
      row = jnp.bitwise_or(
          jnp.bitwise_or(
              jnp.bitwise_and(v, -TCD),
              jnp.left_shift(jnp.bitwise_and(v, SPAN - 1), 1)),
          jnp.bitwise_and(jnp.right_shift(v, SPAN.bit_length() - 1), 1))
      idx_ref[pl.ds(k * 16, 16)] = row
      return carry

    lax.fori_loop(0, n // 16, step, 0)

  def copy_idx(c, buf, sem=None):
    for k in range(CHUNK):
      src = neg_hbm.at[base + c * CHUNK + k]
      dst = negidx[buf].at[pl.ds(k * NEG, NEG)]
      if sem is None:
        pltpu.sync_copy(src, dst)
      else:
        pltpu.async_copy(src, dst, sem)

  def wait_idx(c, buf, sem):
    for k in range(CHUNK):
      pltpu.make_async_copy(neg_hbm.at[base + c * CHUNK + k],
                            negidx[buf].at[pl.ds(k * NEG, NEG)], sem).wait()

  def out_slice(c):
    return neg_out.at[pl.ds((base + c * CHUNK) * NEG, RPC)]

  def start_rows(buf):
    for j in range(NGATH):
      pltpu.async_copy(
          ent_hbm.at[negidx[buf].at[pl.ds(j * 128, 128)]],
          negrow[buf].at[pl.ds(j * 128, 128), :],
          rsem[buf])

  def wait_rows(buf):
    for j in range(NGATH):
      pltpu.make_async_copy(
          ent_hbm.at[negidx[buf].at[pl.ds(j * 128, 128)]],
          negrow[buf].at[pl.ds(j * 128, 128), :],
          rsem[buf]).wait()

  pltpu.sync_copy(tgt_hbm.at[pl.ds(base, BPW)], tgt_v)
  pltpu.sync_copy(pos_hbm.at[pl.ds(base, BPW)], posidx_v)
  pltpu.sync_copy(x_hbm.at[tgt_v], pred_v)
  transform(posidx_v, BPW)
  pltpu.sync_copy(ent_hbm.at[posidx_v], posrow_v)

  copy_idx(0, 0)
  transform(negidx0, RPC)
  start_rows(0)
  copy_idx(1, 1, isem[1])

  def pos_group(g, carry):
    possub = posrow_v.at[pl.ds(g * 16, 16), :]
    predsub = pred_v.at[pl.ds(g * 16, 16), :]

    def d0_body(d0, acc, possub=possub, predsub=predsub):
      cols = jnp.bitwise_and(iota + d0, DIM - 1)
      ev = plsc.load_gather(possub, [iota, cols])
      pv = plsc.load_gather(predsub, [iota, cols])
      return acc + jnp.abs(ev - pv)

    acc = lax.fori_loop(0, DIM, d0_body, jnp.zeros((16,), jnp.float32),
                        unroll=8)
    posout_v[pl.ds(g * 16, 16)] = GAMMA - acc
    return carry

  lax.fori_loop(0, BPW // 16, pos_group, 0)
  pltpu.sync_copy(posout_v, pos_out.at[pl.ds(base, BPW)])

  def compute_chunk(c, buf):
    for bb in range(CHUNK):
      prow = c * CHUNK + bb
      pred_row = pred_v.at[prow]

      def d0_body(d0, accs, pred_row=pred_row, bb=bb, buf=buf):
        cols = jnp.bitwise_and(iota + d0, DIM - 1)
        prot = plsc.load_gather(pred_row, [cols])
        new = []
        for g in range(16):
          sub = negrow[buf].at[pl.ds(bb * NEG + g * 16, 16), :]
          ev = plsc.load_gather(sub, [iota, cols])
          new.append(accs[g] + jnp.abs(ev - prot))
        return tuple(new)

      accs = lax.fori_loop(0, DIM, d0_body,
                           (jnp.zeros((16,), jnp.float32),) * 16,
                           unroll=16)
      for g in range(16):
        negout[buf][pl.ds(bb * NEG + g * 16, 16)] = GAMMA - accs[g]

  def handle(c, buf):
    wait_rows(buf)

    nbuf = 1 - buf

    @pl.when(c + 1 < NCHUNKS)
    def _():
      wait_idx(c + 1, nbuf, isem[nbuf])
      transform(negidx[nbuf], RPC)
      start_rows(nbuf)

    @pl.when(c + 2 < NCHUNKS)
    def _():
      copy_idx(c + 2, buf, isem[buf])

    @pl.when(c >= 2)
    def _():
      pltpu.make_async_copy(negout[buf], out_slice(c - 2), osem[buf]).wait()

    compute_chunk(c, buf)
    pltpu.async_copy(negout[buf], out_slice(c), osem[buf])

  def pair_body(p, carry):
    handle(2 * p, 0)
    handle(2 * p + 1, 1)
    return carry

  lax.fori_loop(0, NCHUNKS // 2, pair_body, 0)

  pltpu.make_async_copy(negout[0], out_slice(NCHUNKS - 2), osem[0]).wait()
  pltpu.make_async_copy(negout[1], out_slice(NCHUNKS - 1), osem[1]).wait()


@jax.jit
def _full(x, tgt, pos, neg, ent):
  ent_rows = _tc_relayout(ent.T).reshape(NROWS64, DIM)
  mesh = plsc.VectorSubcoreMesh(core_axis_name="c", subcore_axis_name="s")
  f = functools.partial(
      pl.kernel,
      mesh=mesh,
      compiler_params=pltpu.CompilerParams(
          needs_layout_passes=False, use_tc_tiling_on_sc=False),
      out_type=(
          jax.ShapeDtypeStruct((BATCH,), jnp.float32),
          jax.ShapeDtypeStruct((BATCH * NEG,), jnp.float32),
      ),
      scratch_types=[
          pltpu.VMEM((BPW,), jnp.int32),
          pltpu.VMEM((BPW,), jnp.int32),
          pltpu.VMEM((RPC,), jnp.int32),
          pltpu.VMEM((RPC,), jnp.int32),
          pltpu.VMEM((BPW, DIM), jnp.float32),
          pltpu.VMEM((BPW, DIM), jnp.float32),
          pltpu.VMEM((RPC, DIM), jnp.float32),
          pltpu.VMEM((RPC, DIM), jnp.float32),
          pltpu.VMEM((BPW,), jnp.float32),
          pltpu.VMEM((RPC,), jnp.float32),
          pltpu.VMEM((RPC,), jnp.float32),
          pltpu.SemaphoreType.DMA,
          pltpu.SemaphoreType.DMA,
          pltpu.SemaphoreType.DMA,
          pltpu.SemaphoreType.DMA,
          pltpu.SemaphoreType.DMA,
          pltpu.SemaphoreType.DMA,
      ],
  )(_sc_body)
  return f(x, tgt, pos, neg, ent_rows)


def kernel(x, target_node_idxes, positive_samples, negative_samples,
           ent_embedding):
  tgt = target_node_idxes.astype(jnp.int32)
  pos = positive_samples.astype(jnp.int32)
  neg = negative_samples.astype(jnp.int32)
  pos_l, neg_l = _full(x, tgt, pos, neg, ent_embedding)
  return pos_l[:, None], neg_l.reshape(BATCH, NEG)
